# Initial kernel scaffold; baseline (speedup 1.0000x reference)
import jax
import jax.numpy as jnp
from jax.experimental import pallas as pl


def kernel(x, edge_index, proj0_W, proj0_b, proj1_W, proj1_b, att_src_00, att_dst_00, att_src_11, att_dst_11, att_src_01, att_dst_01, att_src_10, att_dst_10, k_lin_W, k_lin_b, q, lin_W, lin_b, fc_W, fc_b):
    raise NotImplementedError("write your pallas kernel here")


# speedup vs baseline: 62.5264x; 62.5264x over previous
import functools

import jax
import jax.numpy as jnp
from jax import lax
from jax.experimental import pallas as pl
from jax.experimental.pallas import tpu as pltpu
from jax.experimental.pallas import tpu_sc as plsc

N = 10000
E = 320000
H = 8
HD = 16
RW = 144
ADW = 16
NC = 2
NS = 16
EPS = E // NS
C = 80
NCHUNK = EPS // C
BLK = 1000
NBLK = N // BLK
RPS = N // NS


def _k1_body(x_ref, p0w_ref, p0b_ref, p1w_ref, p1b_ref,
             as0_ref, ad0_ref, as1_ref, ad1_ref, hext_ref, adt_ref):
    xb = x_ref[...]
    B = xb.shape[0]
    h0 = jnp.dot(xb[:, 0:64], p0w_ref[...],
                 preferred_element_type=jnp.float32) + p0b_ref[...]
    h1 = jnp.dot(xb[:, 64:127], p1w_ref[...],
                 preferred_element_type=jnp.float32) + p1b_ref[...]

    def logits(h, a):
        return (h * a).reshape(B, H, HD).sum(-1)

    z8 = jnp.zeros((B, 8), jnp.float32)
    hext_ref[0] = jnp.concatenate([h0, logits(h0, as0_ref[...]), z8], axis=1)
    hext_ref[1] = jnp.concatenate([h1, logits(h1, as1_ref[...]), z8], axis=1)
    adt_ref[0] = jnp.concatenate([logits(h0, ad0_ref[...]), z8], axis=1)
    adt_ref[1] = jnp.concatenate([logits(h1, ad1_ref[...]), z8], axis=1)


_k1_call = pl.pallas_call(
    _k1_body,
    grid=(NBLK,),
    in_specs=[
        pl.BlockSpec((BLK, 128), lambda i: (i, 0)),
        pl.BlockSpec((64, 128), lambda i: (0, 0)),
        pl.BlockSpec((1, 128), lambda i: (0, 0)),
        pl.BlockSpec((63, 128), lambda i: (0, 0)),
        pl.BlockSpec((1, 128), lambda i: (0, 0)),
        pl.BlockSpec((1, 128), lambda i: (0, 0)),
        pl.BlockSpec((1, 128), lambda i: (0, 0)),
        pl.BlockSpec((1, 128), lambda i: (0, 0)),
        pl.BlockSpec((1, 128), lambda i: (0, 0)),
    ],
    out_specs=[
        pl.BlockSpec((2, BLK, RW), lambda i: (0, i, 0)),
        pl.BlockSpec((2, BLK, ADW), lambda i: (0, i, 0)),
    ],
    out_shape=[
        jax.ShapeDtypeStruct((2, N, RW), jnp.float32),
        jax.ShapeDtypeStruct((2, N, ADW), jnp.float32),
    ],
)


_sc_mesh = plsc.VectorSubcoreMesh(core_axis_name="c", subcore_axis_name="s")


@functools.partial(
    pl.kernel,
    out_type=jax.ShapeDtypeStruct((2, N, RW), jnp.float32),
    mesh=_sc_mesh,
    scratch_types=[
        pltpu.VMEM_SHARED((N, RW), jnp.float32),
        pltpu.VMEM((C,), jnp.int32),
        pltpu.VMEM((C,), jnp.int32),
        pltpu.VMEM((C,), jnp.int32),
        pltpu.VMEM((C,), jnp.int32),
        pltpu.VMEM((C, RW), jnp.float32),
        pltpu.VMEM((C, ADW), jnp.float32),
        pltpu.SemaphoreType.DMA,
        pltpu.SemaphoreType.DMA,
    ],
    compiler_params=pltpu.CompilerParams(use_tc_tiling_on_sc=False),
)
def _sc_edges(src_hbm, dst_hbm, hext_hbm, adt_hbm, out_hbm,
              acc_sh, src_v, dst_v, srcoff_v, dstoff_v, rows_v, ad_v,
              sem1, sem2):
    c = lax.axis_index("c")
    s = lax.axis_index("s")
    cn = c * N
    row0 = s * RPS

    def zero_row(e, carry):
        for k in range(RW // 16):
            rows_v[e, pl.ds(k * 16, 16)] = jnp.zeros((16,), jnp.float32)
        return carry

    lax.fori_loop(0, C, zero_row, 0)

    nfull = RPS // C
    nrem = RPS - nfull * C

    def zcp(j, carry):
        pltpu.sync_copy(rows_v, acc_sh.at[pl.ds(row0 + j * C, C)])
        return carry

    lax.fori_loop(0, nfull, zcp, 0)
    pltpu.sync_copy(rows_v.at[pl.ds(0, nrem)],
                    acc_sh.at[pl.ds(row0 + nfull * C, nrem)])
    plsc.subcore_barrier()

    ebase = s * EPS

    def chunk_body(t, carry):
        base = ebase + t * C
        pltpu.sync_copy(src_hbm.at[pl.ds(base, C)], src_v)
        pltpu.sync_copy(dst_hbm.at[pl.ds(base, C)], dst_v)

        def adj(g, cr):
            sl = pl.ds(g * 16, 16)
            srcoff_v[sl] = src_v[sl] + cn
            dstoff_v[sl] = dst_v[sl] + cn
            return cr

        lax.fori_loop(0, C // 16, adj, 0)

        cp1 = pltpu.async_copy(hext_hbm.at[srcoff_v], rows_v, sem1)
        cp2 = pltpu.async_copy(adt_hbm.at[dstoff_v], ad_v, sem2)
        cp1.wait()
        cp2.wait()

        def edge_body(e, cr):
            asrc = rows_v[e, pl.ds(128, 16)]
            adst = ad_v[e, pl.ds(0, 16)]
            z = asrc + adst
            w = jnp.exp(jnp.maximum(z, 0.2 * z))
            rows_v[e, pl.ds(128, 16)] = w
            for k in range(H):
                wk = w[k]
                sl = pl.ds(k * HD, HD)
                rows_v[e, sl] = rows_v[e, sl] * wk
            return cr

        lax.fori_loop(0, C, edge_body, 0)
        pltpu.sync_copy(rows_v, acc_sh.at[dst_v], add=True)
        return carry

    lax.fori_loop(0, NCHUNK, chunk_body, 0)
    plsc.subcore_barrier()

    pltpu.sync_copy(acc_sh.at[pl.ds(row0, RPS)],
                    out_hbm.at[c, pl.ds(row0, RPS)])


def _k2_body(acc_ref, hext_ref, kW_ref, kb_ref, q_ref, lW_ref, lb_ref,
             fW_ref, fb_ref, out_ref, score_ref):
    p = pl.program_id(0)
    i = pl.program_id(1)
    a = acc_ref[...]
    hx = hext_ref[...]
    B = a.shape[1]

    def norm(t):
        acc = a[t, :, 0:128]
        den = a[t, :, 128:136]
        return jnp.maximum(
            (acc.reshape(B, H, HD) / (den[:, :, None] + 1e-16)).reshape(B, 128),
            0.0)

    f00 = norm(0)
    f11 = norm(1)
    f01 = jnp.maximum(hx[0, :, 0:128], 0.0)
    f10 = jnp.maximum(hx[1, :, 0:128], 0.0)
    kW = kW_ref[...]
    kb = kb_ref[...]

    @pl.when(p == 0)
    def _phase0():
        @pl.when(i == 0)
        def _init():
            score_ref[...] = jnp.zeros_like(score_ref)

        for idx, f in ((0, f00), (1, f10), (2, f11), (3, f01)):
            part = jnp.sum(
                jnp.tanh(jnp.dot(f, kW, preferred_element_type=jnp.float32)
                         + kb),
                axis=0, keepdims=True)
            score_ref[pl.ds(idx, 1), :] += part

    @pl.when(p == 1)
    def _phase1():
        t4 = score_ref[...] * (1.0 / N)
        sv = jnp.sum(q_ref[...] * t4, axis=1)
        ev = jnp.exp(sv)
        w00 = ev[0] / (ev[0] + ev[1])
        w10 = 1.0 - w00
        w11 = ev[2] / (ev[2] + ev[3])
        w01 = 1.0 - w11
        lW = lW_ref[...]
        lb = lb_ref[...]
        agg0 = w00 * f00 + w10 * f10
        agg1 = w11 * f11 + w01 * f01
        o0 = jnp.dot(agg0, lW, preferred_element_type=jnp.float32) + lb
        o1 = jnp.dot(agg1, lW, preferred_element_type=jnp.float32) + lb
        ob = jnp.concatenate([o0, o1], axis=1)
        out_ref[...] = (jnp.dot(ob, fW_ref[...],
                                preferred_element_type=jnp.float32)
                        + fb_ref[...])


_k2_call = pl.pallas_call(
    _k2_body,
    grid=(2, NBLK),
    in_specs=[
        pl.BlockSpec((2, BLK, RW), lambda p, i: (0, i, 0)),
        pl.BlockSpec((2, BLK, RW), lambda p, i: (0, i, 0)),
        pl.BlockSpec((128, 128), lambda p, i: (0, 0)),
        pl.BlockSpec((1, 128), lambda p, i: (0, 0)),
        pl.BlockSpec((1, 128), lambda p, i: (0, 0)),
        pl.BlockSpec((128, 128), lambda p, i: (0, 0)),
        pl.BlockSpec((1, 128), lambda p, i: (0, 0)),
        pl.BlockSpec((256, 64), lambda p, i: (0, 0)),
        pl.BlockSpec((1, 64), lambda p, i: (0, 0)),
    ],
    out_specs=pl.BlockSpec((BLK, 64), lambda p, i: (i, 0)),
    out_shape=jax.ShapeDtypeStruct((N, 64), jnp.float32),
    scratch_shapes=[pltpu.VMEM((4, 128), jnp.float32)],
)


def kernel(x, edge_index, proj0_W, proj0_b, proj1_W, proj1_b,
           att_src_00, att_dst_00, att_src_11, att_dst_11,
           att_src_01, att_dst_01, att_src_10, att_dst_10,
           k_lin_W, k_lin_b, q, lin_W, lin_b, fc_W, fc_b):
    src = edge_index[0]
    dst = edge_index[1]
    hext2, adt2 = _k1_call(
        x, proj0_W, proj0_b.reshape(1, 128), proj1_W, proj1_b.reshape(1, 128),
        att_src_00.reshape(1, 128), att_dst_00.reshape(1, 128),
        att_src_11.reshape(1, 128), att_dst_11.reshape(1, 128))
    hext = hext2.reshape(2 * N, RW)
    adt = adt2.reshape(2 * N, ADW)
    accout = _sc_edges(src, dst, hext, adt)
    out = _k2_call(accout, hext2, k_lin_W, k_lin_b.reshape(1, 128),
                   q.reshape(1, 128), lin_W, lin_b.reshape(1, 128),
                   fc_W, fc_b.reshape(1, 64))
    return out


# speedup vs baseline: 162.6111x; 2.6007x over previous
import functools

import jax
import jax.numpy as jnp
from jax import lax
from jax.experimental import pallas as pl
from jax.experimental.pallas import tpu as pltpu
from jax.experimental.pallas import tpu_sc as plsc

N = 10000
E = 320000
H = 8
HD = 16
ADW = 16
NC = 2
NS = 16
EPS = E // NS
C = 80
NCHUNK = EPS // C
BLK = 1000
NBLK = N // BLK
RPS = N // NS


def _head_selector(off):
    j = lax.broadcasted_iota(jnp.int32, (16, 128), 0)
    l = lax.broadcasted_iota(jnp.int32, (16, 128), 1)
    return jnp.where(l // HD == j - off, 1.0, 0.0).astype(jnp.float32)


def _head_selector_t(off):
    l = lax.broadcasted_iota(jnp.int32, (128, 16), 0)
    j = lax.broadcasted_iota(jnp.int32, (128, 16), 1)
    return jnp.where(j == l // HD + off, 1.0, 0.0).astype(jnp.float32)


def _k1_body(x_ref, p0w_ref, p1w_ref, pb_ref,
             as0_ref, ad0_ref, as1_ref, ad1_ref, kW_ref, kb_ref,
             h_ref, asad_ref, tsc_ref, acc_ref):
    t = pl.program_id(0)
    i = pl.program_id(1)
    xb = x_ref[...]
    xs = jnp.where(t == 0, xb[:, 0:64], xb[:, 64:128])
    Ws = jnp.where(t == 0, p0w_ref[...], p1w_ref[...])
    pb = jnp.where(t == 0, pb_ref[0:1, :], pb_ref[1:2, :])
    h = jnp.dot(xs, Ws, preferred_element_type=jnp.float32) + pb
    a_s = jnp.where(t == 0, as0_ref[...], as1_ref[...])
    a_d = jnp.where(t == 0, ad0_ref[...], ad1_ref[...])
    Es = _head_selector_t(0)
    Ed = _head_selector_t(H)
    asad = (jnp.dot(h * a_s, Es, preferred_element_type=jnp.float32)
            + jnp.dot(h * a_d, Ed, preferred_element_type=jnp.float32))
    h_ref[...] = h
    asad_ref[...] = asad

    fid = jnp.maximum(h, 0.0)
    part = jnp.sum(
        jnp.tanh(jnp.dot(fid, kW_ref[...],
                         preferred_element_type=jnp.float32) + kb_ref[...]),
        axis=0, keepdims=True)

    @pl.when(jnp.logical_and(t == 0, i == 0))
    def _init():
        acc_ref[...] = jnp.zeros_like(acc_ref)

    @pl.when(t == 0)
    def _a0():
        acc_ref[1:2, :] += part

    @pl.when(t == 1)
    def _a1():
        acc_ref[0:1, :] += part

    @pl.when(jnp.logical_and(t == 1, i == NBLK - 1))
    def _fin():
        tsc_ref[...] = acc_ref[...]


_k1_call = pl.pallas_call(
    _k1_body,
    grid=(2, NBLK),
    in_specs=[
        pl.BlockSpec((BLK, 128), lambda t, i: (i, 0)),
        pl.BlockSpec((64, 128), lambda t, i: (0, 0)),
        pl.BlockSpec((64, 128), lambda t, i: (0, 0)),
        pl.BlockSpec((2, 128), lambda t, i: (0, 0)),
        pl.BlockSpec((1, 128), lambda t, i: (0, 0)),
        pl.BlockSpec((1, 128), lambda t, i: (0, 0)),
        pl.BlockSpec((1, 128), lambda t, i: (0, 0)),
        pl.BlockSpec((1, 128), lambda t, i: (0, 0)),
        pl.BlockSpec((128, 128), lambda t, i: (0, 0)),
        pl.BlockSpec((1, 128), lambda t, i: (0, 0)),
    ],
    out_specs=[
        pl.BlockSpec((BLK, 128), lambda t, i: (t * NBLK + i, 0)),
        pl.BlockSpec((BLK, ADW), lambda t, i: (t * NBLK + i, 0)),
        pl.BlockSpec((2, 128), lambda t, i: (0, 0)),
    ],
    out_shape=[
        jax.ShapeDtypeStruct((2 * N, 128), jnp.float32),
        jax.ShapeDtypeStruct((2 * N, ADW), jnp.float32),
        jax.ShapeDtypeStruct((2, 128), jnp.float32),
    ],
    scratch_shapes=[pltpu.VMEM((2, 128), jnp.float32)],
)


_sc_mesh = plsc.VectorSubcoreMesh(core_axis_name="c", subcore_axis_name="s")

NBUF = 3
_BUF_SCRATCH = []
for _ in range(NBUF):
    _BUF_SCRATCH += [
        pltpu.VMEM((C,), jnp.int32),
        pltpu.VMEM((C,), jnp.int32),
        pltpu.VMEM((C,), jnp.int32),
        pltpu.VMEM((C,), jnp.int32),
        pltpu.VMEM((C,), jnp.int32),
        pltpu.VMEM((C, 128), jnp.float32),
        pltpu.VMEM((C, ADW), jnp.float32),
        pltpu.VMEM((C, ADW), jnp.float32),
        pltpu.SemaphoreType.DMA,
        pltpu.SemaphoreType.DMA,
        pltpu.SemaphoreType.DMA,
        pltpu.SemaphoreType.DMA,
        pltpu.SemaphoreType.DMA,
        pltpu.SemaphoreType.DMA,
    ]
_NB = 14


@functools.partial(
    pl.kernel,
    out_type=[
        jax.ShapeDtypeStruct((2 * N, 128), jnp.float32),
        jax.ShapeDtypeStruct((2 * N, ADW), jnp.float32),
    ],
    mesh=_sc_mesh,
    scratch_types=[
        pltpu.VMEM_SHARED((N, 128), jnp.float32),
        pltpu.VMEM_SHARED((N, ADW), jnp.float32),
    ] + _BUF_SCRATCH,
    compiler_params=pltpu.CompilerParams(use_tc_tiling_on_sc=False,
                                         needs_layout_passes=False),
)
def _sc_edges(ei_hbm, h_hbm, asad_hbm, msg_hbm, den_hbm,
              accm_sh, accd_sh, *bufs):
    c = lax.axis_index("c")
    s = lax.axis_index("s")
    cn = c * N
    row0 = s * RPS
    B = [bufs[i * _NB:(i + 1) * _NB] for i in range(NBUF)]

    def issue_idx(t, b):
        base = ebase + t * C
        pltpu.async_copy(ei_hbm.at[0, pl.ds(base, C)], B[b][0], B[b][8])
        pltpu.async_copy(ei_hbm.at[1, pl.ds(base, C)], B[b][1], B[b][8])

    def wait_idx(b):
        pltpu.make_async_copy(ei_hbm.at[0, pl.ds(0, C)], B[b][0],
                              B[b][8]).wait()
        pltpu.make_async_copy(ei_hbm.at[1, pl.ds(0, C)], B[b][1],
                              B[b][8]).wait()

    def adjust(b):
        src_v, dst_v, soff, doff, draw = B[b][:5]

        def adj(g, cr):
            sl = pl.ds(g * 16, 16)
            sv = src_v[sl]
            dv = dst_v[sl]
            soff[sl] = sv + cn
            doff[sl] = dv + cn
            draw[sl] = dv
            return cr

        lax.fori_loop(0, C // 16, adj, 0)

    def issue_gather(b):
        pltpu.async_copy(h_hbm.at[B[b][2]], B[b][5], B[b][9])
        pltpu.async_copy(asad_hbm.at[B[b][2]], B[b][6], B[b][10])
        pltpu.async_copy(asad_hbm.at[B[b][3]], B[b][7], B[b][11])

    def wait_gather(b):
        pltpu.make_async_copy(h_hbm.at[B[b][2]], B[b][5], B[b][9]).wait()
        pltpu.make_async_copy(asad_hbm.at[B[b][2]], B[b][6],
                              B[b][10]).wait()
        pltpu.make_async_copy(asad_hbm.at[B[b][3]], B[b][7],
                              B[b][11]).wait()

    def issue_scatter(b):
        pltpu.async_copy(B[b][5], accm_sh.at[B[b][4]], B[b][12], add=True)
        pltpu.async_copy(B[b][7], accd_sh.at[B[b][4]], B[b][13], add=True)

    def wait_scatter(b):
        pltpu.make_async_copy(B[b][5], accm_sh.at[B[b][4]], B[b][12]).wait()
        pltpu.make_async_copy(B[b][7], accd_sh.at[B[b][4]], B[b][13]).wait()

    def compute(b):
        rows, asv, adv = B[b][5], B[b][6], B[b][7]

        def wgroup(g, cr):
            eg = g * 16 + lax.iota(jnp.int32, 16)
            for k in range(H):
                as_k = plsc.load_gather(
                    asv, [eg, jnp.full((16,), k, jnp.int32)])
                ad_k = plsc.load_gather(
                    adv, [eg, jnp.full((16,), H + k, jnp.int32)])
                z = as_k + ad_k
                w_k = jnp.exp(jnp.maximum(z, 0.2 * z))
                plsc.store_scatter(
                    adv, [eg, jnp.full((16,), k, jnp.int32)], w_k)
            return cr

        lax.fori_loop(0, C // 16, wgroup, 0)

        def edge_body(e, cr):
            wv = adv[e, pl.ds(0, 16)]
            for k in range(H):
                sl = pl.ds(k * HD, HD)
                rows[e, sl] = rows[e, sl] * wv[k]
            return cr

        lax.fori_loop(0, C, edge_body, 0)

    rows0_v = B[0][5]

    def zero_row(e, cr):
        for k in range(128 // 16):
            rows0_v[e, pl.ds(k * 16, 16)] = jnp.zeros((16,), jnp.float32)
        return cr

    lax.fori_loop(0, C, zero_row, 0)

    nfull = RPS // C
    nrem = RPS - nfull * C

    def zcp(j, cr):
        pltpu.sync_copy(rows0_v, accm_sh.at[pl.ds(row0 + j * C, C)])
        return cr

    lax.fori_loop(0, nfull, zcp, 0)
    pltpu.sync_copy(rows0_v.at[pl.ds(0, nrem)],
                    accm_sh.at[pl.ds(row0 + nfull * C, nrem)])

    def zcpd(j, cr):
        pltpu.sync_copy(rows0_v.at[pl.ds(0, C), pl.ds(0, ADW)],
                        accd_sh.at[pl.ds(row0 + j * C, C)])
        return cr

    lax.fori_loop(0, nfull, zcpd, 0)
    pltpu.sync_copy(rows0_v.at[pl.ds(0, nrem), pl.ds(0, ADW)],
                    accd_sh.at[pl.ds(row0 + nfull * C, nrem)])
    plsc.subcore_barrier()

    ebase = s * EPS
    issue_idx(0, 0)
    wait_idx(0)
    adjust(0)
    issue_gather(0)
    issue_idx(1, 1)
    issue_idx(2, 2)

    NJ = NCHUNK // NBUF

    def loop_j(j, carry):
        t0 = j * NBUF
        for u in range(NBUF):
            t = t0 + u
            b = u
            bb = (u + 1) % NBUF
            wait_idx(bb)

            @pl.when(t >= 2)
            def _ws():
                wait_scatter(bb)

            adjust(bb)
            issue_gather(bb)

            @pl.when(t + 2 < NCHUNK)
            def _ii():
                issue_idx(t + 2, (u + 2) % NBUF)

            wait_gather(b)
            compute(b)
            issue_scatter(b)
        return carry

    lax.fori_loop(0, NJ, loop_j, 0)
    wait_gather(0)
    compute(0)
    issue_scatter(0)

    wait_scatter(1)
    wait_scatter(2)
    wait_scatter(0)
    plsc.subcore_barrier()

    pltpu.sync_copy(accm_sh.at[pl.ds(row0, RPS)],
                    msg_hbm.at[pl.ds(cn + row0, RPS)])
    pltpu.sync_copy(accd_sh.at[pl.ds(row0, RPS)],
                    den_hbm.at[pl.ds(cn + row0, RPS)])


def _k2_body(m0_ref, m1_ref, d0_ref, d1_ref, h0_ref, h1_ref, tsc_ref,
             kW_ref, kb_ref, q_ref, lW_ref, lb_ref, fW_ref, fb_ref,
             out_ref, score_ref):
    p = pl.program_id(0)
    i = pl.program_id(1)
    Eb = _head_selector(0)

    def norm(m_ref, d_ref):
        den128 = jnp.dot(d_ref[...], Eb, preferred_element_type=jnp.float32)
        return jnp.maximum(m_ref[...] / (den128 + 1e-16), 0.0)

    f00 = norm(m0_ref, d0_ref)
    f11 = norm(m1_ref, d1_ref)
    kW = kW_ref[...]
    kb = kb_ref[...]

    @pl.when(p == 0)
    def _phase0():
        @pl.when(i == 0)
        def _init():
            score_ref[...] = jnp.zeros_like(score_ref)

        p00 = jnp.sum(
            jnp.tanh(jnp.dot(f00, kW, preferred_element_type=jnp.float32)
                     + kb), axis=0, keepdims=True)
        p11 = jnp.sum(
            jnp.tanh(jnp.dot(f11, kW, preferred_element_type=jnp.float32)
                     + kb), axis=0, keepdims=True)
        score_ref[0:1, :] += p00
        score_ref[1:2, :] += p11

    @pl.when(p == 1)
    def _phase1():
        f01 = jnp.maximum(h0_ref[...], 0.0)
        f10 = jnp.maximum(h1_ref[...], 0.0)
        qv = q_ref[...] * (1.0 / N)
        s00 = jnp.sum(qv * score_ref[0:1, :])
        s11 = jnp.sum(qv * score_ref[1:2, :])
        s10 = jnp.sum(qv * tsc_ref[0:1, :])
        s01 = jnp.sum(qv * tsc_ref[1:2, :])
        e00 = jnp.exp(s00)
        e10 = jnp.exp(s10)
        e11 = jnp.exp(s11)
        e01 = jnp.exp(s01)
        w00 = e00 / (e00 + e10)
        w10 = 1.0 - w00
        w11 = e11 / (e11 + e01)
        w01 = 1.0 - w11
        lW = lW_ref[...]
        lb = lb_ref[...]
        agg0 = w00 * f00 + w10 * f10
        agg1 = w11 * f11 + w01 * f01
        o0 = jnp.dot(agg0, lW, preferred_element_type=jnp.float32) + lb
        o1 = jnp.dot(agg1, lW, preferred_element_type=jnp.float32) + lb
        ob = jnp.concatenate([o0, o1], axis=1)
        out_ref[...] = (jnp.dot(ob, fW_ref[...],
                                preferred_element_type=jnp.float32)
                        + fb_ref[...])


_k2_call = pl.pallas_call(
    _k2_body,
    grid=(2, NBLK),
    in_specs=[
        pl.BlockSpec((BLK, 128), lambda p, i: (i, 0)),
        pl.BlockSpec((BLK, 128), lambda p, i: (NBLK + i, 0)),
        pl.BlockSpec((BLK, ADW), lambda p, i: (i, 0)),
        pl.BlockSpec((BLK, ADW), lambda p, i: (NBLK + i, 0)),
        pl.BlockSpec((BLK, 128), lambda p, i: (i, 0)),
        pl.BlockSpec((BLK, 128), lambda p, i: (NBLK + i, 0)),
        pl.BlockSpec((2, 128), lambda p, i: (0, 0)),
        pl.BlockSpec((128, 128), lambda p, i: (0, 0)),
        pl.BlockSpec((1, 128), lambda p, i: (0, 0)),
        pl.BlockSpec((1, 128), lambda p, i: (0, 0)),
        pl.BlockSpec((128, 128), lambda p, i: (0, 0)),
        pl.BlockSpec((1, 128), lambda p, i: (0, 0)),
        pl.BlockSpec((256, 64), lambda p, i: (0, 0)),
        pl.BlockSpec((1, 64), lambda p, i: (0, 0)),
    ],
    out_specs=pl.BlockSpec((BLK, 64), lambda p, i: (i, 0)),
    out_shape=jax.ShapeDtypeStruct((N, 64), jnp.float32),
    scratch_shapes=[pltpu.VMEM((2, 128), jnp.float32)],
)


def kernel(x, edge_index, proj0_W, proj0_b, proj1_W, proj1_b,
           att_src_00, att_dst_00, att_src_11, att_dst_11,
           att_src_01, att_dst_01, att_src_10, att_dst_10,
           k_lin_W, k_lin_b, q, lin_W, lin_b, fc_W, fc_b):
    p1w = jnp.concatenate([proj1_W, jnp.zeros((1, 128), jnp.float32)], 0)
    pb = jnp.stack([proj0_b, proj1_b])
    h, asad, tsc = _k1_call(
        x, proj0_W, p1w, pb,
        att_src_00.reshape(1, 128), att_dst_00.reshape(1, 128),
        att_src_11.reshape(1, 128), att_dst_11.reshape(1, 128),
        k_lin_W, k_lin_b.reshape(1, 128))
    msg, den = _sc_edges(edge_index, h, asad)
    out = _k2_call(msg, msg, den, den, h, h, tsc,
                   k_lin_W, k_lin_b.reshape(1, 128), q.reshape(1, 128),
                   lin_W, lin_b.reshape(1, 128), fc_W, fc_b.reshape(1, 64))
    return out


# speedup vs baseline: 177.7766x; 1.0933x over previous
import functools

import jax
import jax.numpy as jnp
from jax import lax
from jax.experimental import pallas as pl
from jax.experimental.pallas import tpu as pltpu
from jax.experimental.pallas import tpu_sc as plsc

N = 10000
E = 320000
H = 8
HD = 16
ADW = 16
NC = 2
NS = 16
EPS = E // NS
C = 80
NCHUNK = EPS // C
BLK = 1000
NBLK = N // BLK
RPS = N // NS


def _head_selector(off):
    j = lax.broadcasted_iota(jnp.int32, (16, 128), 0)
    l = lax.broadcasted_iota(jnp.int32, (16, 128), 1)
    return jnp.where(l // HD == j - off, 1.0, 0.0).astype(jnp.float32)


def _head_selector_t(off):
    l = lax.broadcasted_iota(jnp.int32, (128, 16), 0)
    j = lax.broadcasted_iota(jnp.int32, (128, 16), 1)
    return jnp.where(j == l // HD + off, 1.0, 0.0).astype(jnp.float32)


def _k1_body(x_ref, p0w_ref, p1w_ref, pb_ref,
             as0_ref, ad0_ref, as1_ref, ad1_ref, kW_ref, kb_ref,
             h_ref, asad_ref, tsc_ref, acc_ref):
    t = pl.program_id(0)
    i = pl.program_id(1)
    xb = x_ref[...]
    xs = jnp.where(t == 0, xb[:, 0:64], xb[:, 64:128])
    Ws = jnp.where(t == 0, p0w_ref[...], p1w_ref[...])
    pb = jnp.where(t == 0, pb_ref[0:1, :], pb_ref[1:2, :])
    h = jnp.dot(xs, Ws, preferred_element_type=jnp.float32) + pb
    a_s = jnp.where(t == 0, as0_ref[...], as1_ref[...])
    a_d = jnp.where(t == 0, ad0_ref[...], ad1_ref[...])
    Es = _head_selector_t(0)
    Ed = _head_selector_t(H)
    asad = (jnp.dot(h * a_s, Es, preferred_element_type=jnp.float32,
                    precision=lax.Precision.HIGHEST)
            + jnp.dot(h * a_d, Ed, preferred_element_type=jnp.float32,
                      precision=lax.Precision.HIGHEST))
    h_ref[...] = h
    asad_ref[...] = asad

    fid = jnp.maximum(h, 0.0)
    part = jnp.sum(
        jnp.tanh(jnp.dot(fid, kW_ref[...],
                         preferred_element_type=jnp.float32) + kb_ref[...]),
        axis=0, keepdims=True)

    @pl.when(jnp.logical_and(t == 0, i == 0))
    def _init():
        acc_ref[...] = jnp.zeros_like(acc_ref)

    @pl.when(t == 0)
    def _a0():
        acc_ref[1:2, :] += part

    @pl.when(t == 1)
    def _a1():
        acc_ref[0:1, :] += part

    @pl.when(jnp.logical_and(t == 1, i == NBLK - 1))
    def _fin():
        tsc_ref[...] = acc_ref[...]


_k1_call = pl.pallas_call(
    _k1_body,
    grid=(2, NBLK),
    in_specs=[
        pl.BlockSpec((BLK, 128), lambda t, i: (i, 0)),
        pl.BlockSpec((64, 128), lambda t, i: (0, 0)),
        pl.BlockSpec((64, 128), lambda t, i: (0, 0)),
        pl.BlockSpec((2, 128), lambda t, i: (0, 0)),
        pl.BlockSpec((1, 128), lambda t, i: (0, 0)),
        pl.BlockSpec((1, 128), lambda t, i: (0, 0)),
        pl.BlockSpec((1, 128), lambda t, i: (0, 0)),
        pl.BlockSpec((1, 128), lambda t, i: (0, 0)),
        pl.BlockSpec((128, 128), lambda t, i: (0, 0)),
        pl.BlockSpec((1, 128), lambda t, i: (0, 0)),
    ],
    out_specs=[
        pl.BlockSpec((BLK, 128), lambda t, i: (t * NBLK + i, 0)),
        pl.BlockSpec((BLK, ADW), lambda t, i: (t * NBLK + i, 0)),
        pl.BlockSpec((2, 128), lambda t, i: (0, 0)),
    ],
    out_shape=[
        jax.ShapeDtypeStruct((2 * N, 128), jnp.float32),
        jax.ShapeDtypeStruct((2 * N, ADW), jnp.float32),
        jax.ShapeDtypeStruct((2, 128), jnp.float32),
    ],
    scratch_shapes=[pltpu.VMEM((2, 128), jnp.float32)],
)


_sc_mesh = plsc.VectorSubcoreMesh(core_axis_name="c", subcore_axis_name="s")

NBUF = 3
_BUF_SCRATCH = []
for _ in range(NBUF):
    _BUF_SCRATCH += [
        pltpu.VMEM((C,), jnp.int32),
        pltpu.VMEM((C,), jnp.int32),
        pltpu.VMEM((C,), jnp.int32),
        pltpu.VMEM((C,), jnp.int32),
        pltpu.VMEM((C,), jnp.int32),
        pltpu.VMEM((C, 128), jnp.float32),
        pltpu.VMEM((C, ADW), jnp.float32),
        pltpu.VMEM((C, ADW), jnp.float32),
        pltpu.SemaphoreType.DMA,
        pltpu.SemaphoreType.DMA,
        pltpu.SemaphoreType.DMA,
        pltpu.SemaphoreType.DMA,
        pltpu.SemaphoreType.DMA,
        pltpu.SemaphoreType.DMA,
    ]
_NB = 14


@functools.partial(
    pl.kernel,
    out_type=[
        jax.ShapeDtypeStruct((2 * N, 128), jnp.float32),
        jax.ShapeDtypeStruct((2 * N, ADW), jnp.float32),
    ],
    mesh=_sc_mesh,
    scratch_types=[
        pltpu.VMEM_SHARED((N, 128), jnp.float32),
        pltpu.VMEM_SHARED((N, ADW), jnp.float32),
    ] + _BUF_SCRATCH,
    compiler_params=pltpu.CompilerParams(use_tc_tiling_on_sc=False,
                                         needs_layout_passes=False),
)
def _sc_edges(ei_hbm, h_hbm, asad_hbm, msg_hbm, den_hbm,
              accm_sh, accd_sh, *bufs):
    c = lax.axis_index("c")
    s = lax.axis_index("s")
    cn = c * N
    row0 = s * RPS
    B = [bufs[i * _NB:(i + 1) * _NB] for i in range(NBUF)]

    def issue_idx(t, b):
        base = ebase + t * C
        pltpu.async_copy(ei_hbm.at[0, pl.ds(base, C)], B[b][0], B[b][8])
        pltpu.async_copy(ei_hbm.at[1, pl.ds(base, C)], B[b][1], B[b][8])

    def wait_idx(b):
        pltpu.make_async_copy(ei_hbm.at[0, pl.ds(0, C)], B[b][0],
                              B[b][8]).wait()
        pltpu.make_async_copy(ei_hbm.at[1, pl.ds(0, C)], B[b][1],
                              B[b][8]).wait()

    def adjust(b):
        src_v, dst_v, soff, doff, draw = B[b][:5]

        def adj(g, cr):
            sl = pl.ds(g * 16, 16)
            sv = src_v[sl]
            dv = dst_v[sl]
            soff[sl] = sv + cn
            doff[sl] = dv + cn
            draw[sl] = dv
            return cr

        lax.fori_loop(0, C // 16, adj, 0)

    def issue_gather(b):
        pltpu.async_copy(h_hbm.at[B[b][2]], B[b][5], B[b][9])
        pltpu.async_copy(asad_hbm.at[B[b][2]], B[b][6], B[b][10])
        pltpu.async_copy(asad_hbm.at[B[b][3]], B[b][7], B[b][11])

    def wait_gather(b):
        pltpu.make_async_copy(h_hbm.at[B[b][2]], B[b][5], B[b][9]).wait()
        pltpu.make_async_copy(asad_hbm.at[B[b][2]], B[b][6],
                              B[b][10]).wait()
        pltpu.make_async_copy(asad_hbm.at[B[b][3]], B[b][7],
                              B[b][11]).wait()

    def issue_scatter(b):
        pltpu.async_copy(B[b][5], accm_sh.at[B[b][4]], B[b][12], add=True)
        pltpu.async_copy(B[b][7], accd_sh.at[B[b][4]], B[b][13], add=True)

    def wait_scatter(b):
        pltpu.make_async_copy(B[b][5], accm_sh.at[B[b][4]], B[b][12]).wait()
        pltpu.make_async_copy(B[b][7], accd_sh.at[B[b][4]], B[b][13]).wait()

    def compute(b):
        rows, asv, adv = B[b][5], B[b][6], B[b][7]

        def exp_poly(z):
            y = z * 1.4426950408889634
            ni = y.astype(jnp.int32)
            nf = ni.astype(jnp.float32)
            n = ni - jnp.where(y < nf, 1, 0)
            fr = y - n.astype(jnp.float32)
            p = jnp.float32(0.0018943794234292928)
            for ck in (0.008940582529284601, 0.05587655686901505,
                       0.24013169187194985, 0.6931567766988557,
                       0.9999997696337073):
                p = p * fr + jnp.float32(ck)
            bits = (n + 127) << 23
            return p * lax.bitcast_convert_type(bits, jnp.float32)

        def wgroup(g, cr):
            eg = g * 16 + lax.iota(jnp.int32, 16)
            zs = []
            for k in range(H):
                as_k = plsc.load_gather(
                    asv, [eg, jnp.full((16,), k, jnp.int32)])
                ad_k = plsc.load_gather(
                    adv, [eg, jnp.full((16,), H + k, jnp.int32)])
                zs.append(as_k + ad_k)
            ws = [exp_poly(jnp.maximum(z, 0.2 * z)) for z in zs]
            for k in range(H):
                plsc.store_scatter(
                    adv, [eg, jnp.full((16,), k, jnp.int32)], ws[k])
            return cr

        lax.fori_loop(0, C // 16, wgroup, 0)

        def edge_body(e, cr):
            wv = adv[e, pl.ds(0, 16)]
            for k in range(H):
                sl = pl.ds(k * HD, HD)
                rows[e, sl] = rows[e, sl] * wv[k]
            return cr

        lax.fori_loop(0, C, edge_body, 0)

    rows0_v = B[0][5]

    def zero_row(e, cr):
        for k in range(128 // 16):
            rows0_v[e, pl.ds(k * 16, 16)] = jnp.zeros((16,), jnp.float32)
        return cr

    lax.fori_loop(0, C, zero_row, 0)

    nfull = RPS // C
    nrem = RPS - nfull * C

    def zcp(j, cr):
        pltpu.sync_copy(rows0_v, accm_sh.at[pl.ds(row0 + j * C, C)])
        return cr

    lax.fori_loop(0, nfull, zcp, 0)
    pltpu.sync_copy(rows0_v.at[pl.ds(0, nrem)],
                    accm_sh.at[pl.ds(row0 + nfull * C, nrem)])

    def zcpd(j, cr):
        pltpu.sync_copy(rows0_v.at[pl.ds(0, C), pl.ds(0, ADW)],
                        accd_sh.at[pl.ds(row0 + j * C, C)])
        return cr

    lax.fori_loop(0, nfull, zcpd, 0)
    pltpu.sync_copy(rows0_v.at[pl.ds(0, nrem), pl.ds(0, ADW)],
                    accd_sh.at[pl.ds(row0 + nfull * C, nrem)])
    plsc.subcore_barrier()

    ebase = s * EPS
    issue_idx(0, 0)
    wait_idx(0)
    adjust(0)
    issue_gather(0)
    issue_idx(1, 1)
    issue_idx(2, 2)

    NJ = NCHUNK // NBUF

    def loop_j(j, carry):
        t0 = j * NBUF
        for u in range(NBUF):
            t = t0 + u
            b = u
            bb = (u + 1) % NBUF
            wait_idx(bb)

            @pl.when(t >= 2)
            def _ws():
                wait_scatter(bb)

            adjust(bb)
            issue_gather(bb)

            @pl.when(t + 2 < NCHUNK)
            def _ii():
                issue_idx(t + 2, (u + 2) % NBUF)

            wait_gather(b)
            compute(b)
            issue_scatter(b)
        return carry

    lax.fori_loop(0, NJ, loop_j, 0)
    wait_gather(0)
    compute(0)
    issue_scatter(0)

    wait_scatter(1)
    wait_scatter(2)
    wait_scatter(0)
    plsc.subcore_barrier()

    pltpu.sync_copy(accm_sh.at[pl.ds(row0, RPS)],
                    msg_hbm.at[pl.ds(cn + row0, RPS)])
    pltpu.sync_copy(accd_sh.at[pl.ds(row0, RPS)],
                    den_hbm.at[pl.ds(cn + row0, RPS)])


def _k2_body(m0_ref, m1_ref, d0_ref, d1_ref, h0_ref, h1_ref, tsc_ref,
             kW_ref, kb_ref, q_ref, lW_ref, lb_ref, fW_ref, fb_ref,
             out_ref, score_ref):
    p = pl.program_id(0)
    i = pl.program_id(1)
    Eb = _head_selector(0)

    def norm(m_ref, d_ref):
        den128 = jnp.dot(d_ref[...], Eb, preferred_element_type=jnp.float32,
                         precision=lax.Precision.HIGHEST)
        return jnp.maximum(m_ref[...] / (den128 + 1e-16), 0.0)

    f00 = norm(m0_ref, d0_ref)
    f11 = norm(m1_ref, d1_ref)
    kW = kW_ref[...]
    kb = kb_ref[...]

    @pl.when(p == 0)
    def _phase0():
        @pl.when(i == 0)
        def _init():
            score_ref[...] = jnp.zeros_like(score_ref)

        p00 = jnp.sum(
            jnp.tanh(jnp.dot(f00, kW, preferred_element_type=jnp.float32)
                     + kb), axis=0, keepdims=True)
        p11 = jnp.sum(
            jnp.tanh(jnp.dot(f11, kW, preferred_element_type=jnp.float32)
                     + kb), axis=0, keepdims=True)
        score_ref[0:1, :] += p00
        score_ref[1:2, :] += p11

    @pl.when(p == 1)
    def _phase1():
        f01 = jnp.maximum(h0_ref[...], 0.0)
        f10 = jnp.maximum(h1_ref[...], 0.0)
        qv = q_ref[...] * (1.0 / N)
        s00 = jnp.sum(qv * score_ref[0:1, :])
        s11 = jnp.sum(qv * score_ref[1:2, :])
        s10 = jnp.sum(qv * tsc_ref[0:1, :])
        s01 = jnp.sum(qv * tsc_ref[1:2, :])
        e00 = jnp.exp(s00)
        e10 = jnp.exp(s10)
        e11 = jnp.exp(s11)
        e01 = jnp.exp(s01)
        w00 = e00 / (e00 + e10)
        w10 = 1.0 - w00
        w11 = e11 / (e11 + e01)
        w01 = 1.0 - w11
        lW = lW_ref[...]
        lb = lb_ref[...]
        agg0 = w00 * f00 + w10 * f10
        agg1 = w11 * f11 + w01 * f01
        o0 = jnp.dot(agg0, lW, preferred_element_type=jnp.float32) + lb
        o1 = jnp.dot(agg1, lW, preferred_element_type=jnp.float32) + lb
        ob = jnp.concatenate([o0, o1], axis=1)
        out_ref[...] = (jnp.dot(ob, fW_ref[...],
                                preferred_element_type=jnp.float32)
                        + fb_ref[...])


_k2_call = pl.pallas_call(
    _k2_body,
    grid=(2, NBLK),
    in_specs=[
        pl.BlockSpec((BLK, 128), lambda p, i: (i, 0)),
        pl.BlockSpec((BLK, 128), lambda p, i: (NBLK + i, 0)),
        pl.BlockSpec((BLK, ADW), lambda p, i: (i, 0)),
        pl.BlockSpec((BLK, ADW), lambda p, i: (NBLK + i, 0)),
        pl.BlockSpec((BLK, 128), lambda p, i: (i, 0)),
        pl.BlockSpec((BLK, 128), lambda p, i: (NBLK + i, 0)),
        pl.BlockSpec((2, 128), lambda p, i: (0, 0)),
        pl.BlockSpec((128, 128), lambda p, i: (0, 0)),
        pl.BlockSpec((1, 128), lambda p, i: (0, 0)),
        pl.BlockSpec((1, 128), lambda p, i: (0, 0)),
        pl.BlockSpec((128, 128), lambda p, i: (0, 0)),
        pl.BlockSpec((1, 128), lambda p, i: (0, 0)),
        pl.BlockSpec((256, 64), lambda p, i: (0, 0)),
        pl.BlockSpec((1, 64), lambda p, i: (0, 0)),
    ],
    out_specs=pl.BlockSpec((BLK, 64), lambda p, i: (i, 0)),
    out_shape=jax.ShapeDtypeStruct((N, 64), jnp.float32),
    scratch_shapes=[pltpu.VMEM((2, 128), jnp.float32)],
)


def kernel(x, edge_index, proj0_W, proj0_b, proj1_W, proj1_b,
           att_src_00, att_dst_00, att_src_11, att_dst_11,
           att_src_01, att_dst_01, att_src_10, att_dst_10,
           k_lin_W, k_lin_b, q, lin_W, lin_b, fc_W, fc_b):
    p1w = jnp.concatenate([proj1_W, jnp.zeros((1, 128), jnp.float32)], 0)
    pb = jnp.stack([proj0_b, proj1_b])
    h, asad, tsc = _k1_call(
        x, proj0_W, p1w, pb,
        att_src_00.reshape(1, 128), att_dst_00.reshape(1, 128),
        att_src_11.reshape(1, 128), att_dst_11.reshape(1, 128),
        k_lin_W, k_lin_b.reshape(1, 128))
    msg, den = _sc_edges(edge_index, h, asad)
    out = _k2_call(msg, msg, den, den, h, h, tsc,
                   k_lin_W, k_lin_b.reshape(1, 128), q.reshape(1, 128),
                   lin_W, lin_b.reshape(1, 128), fc_W, fc_b.reshape(1, 64))
    return out


# speedup vs baseline: 193.3375x; 1.0875x over previous
import functools

import jax
import jax.numpy as jnp
from jax import lax
from jax.experimental import pallas as pl
from jax.experimental.pallas import tpu as pltpu
from jax.experimental.pallas import tpu_sc as plsc

N = 10000
E = 320000
H = 8
HD = 16
ADW = 16
NC = 2
NS = 16
EPS = E // NS
C = 80
NCHUNK = EPS // C
BLK = 1000
NBLK = N // BLK
RPS = N // NS


def _head_selector(off):
    j = lax.broadcasted_iota(jnp.int32, (16, 128), 0)
    l = lax.broadcasted_iota(jnp.int32, (16, 128), 1)
    return jnp.where(l // HD == j - off, 1.0, 0.0).astype(jnp.float32)


def _head_selector_t(off):
    l = lax.broadcasted_iota(jnp.int32, (128, 16), 0)
    j = lax.broadcasted_iota(jnp.int32, (128, 16), 1)
    return jnp.where(j == l // HD + off, 1.0, 0.0).astype(jnp.float32)


def _k1_body(x_ref, p0w_ref, p1w_ref, pb_ref,
             as0_ref, ad0_ref, as1_ref, ad1_ref, kW_ref, kb_ref,
             h_ref, asad_ref, tsc_ref, acc_ref):
    t = pl.program_id(0)
    i = pl.program_id(1)
    xb = x_ref[...]
    xs = jnp.where(t == 0, xb[:, 0:64], xb[:, 64:128])
    Ws = jnp.where(t == 0, p0w_ref[...], p1w_ref[...])
    pb = jnp.where(t == 0, pb_ref[0:1, :], pb_ref[1:2, :])
    h = jnp.dot(xs, Ws, preferred_element_type=jnp.float32) + pb
    a_s = jnp.where(t == 0, as0_ref[...], as1_ref[...])
    a_d = jnp.where(t == 0, ad0_ref[...], ad1_ref[...])
    Es = _head_selector_t(0)
    Ed = _head_selector_t(H)
    asad = (jnp.dot(h * a_s, Es, preferred_element_type=jnp.float32)
            + jnp.dot(h * a_d, Ed, preferred_element_type=jnp.float32))
    h_ref[...] = h
    asad_ref[...] = asad

    fid = jnp.maximum(h, 0.0)
    part = jnp.sum(
        jnp.tanh(jnp.dot(fid, kW_ref[...],
                         preferred_element_type=jnp.float32) + kb_ref[...]),
        axis=0, keepdims=True)

    @pl.when(jnp.logical_and(t == 0, i == 0))
    def _init():
        acc_ref[...] = jnp.zeros_like(acc_ref)

    @pl.when(t == 0)
    def _a0():
        acc_ref[1:2, :] += part

    @pl.when(t == 1)
    def _a1():
        acc_ref[0:1, :] += part

    @pl.when(jnp.logical_and(t == 1, i == NBLK - 1))
    def _fin():
        tsc_ref[...] = acc_ref[...]


_k1_call = pl.pallas_call(
    _k1_body,
    grid=(2, NBLK),
    in_specs=[
        pl.BlockSpec((BLK, 128), lambda t, i: (i, 0)),
        pl.BlockSpec((64, 128), lambda t, i: (0, 0)),
        pl.BlockSpec((64, 128), lambda t, i: (0, 0)),
        pl.BlockSpec((2, 128), lambda t, i: (0, 0)),
        pl.BlockSpec((1, 128), lambda t, i: (0, 0)),
        pl.BlockSpec((1, 128), lambda t, i: (0, 0)),
        pl.BlockSpec((1, 128), lambda t, i: (0, 0)),
        pl.BlockSpec((1, 128), lambda t, i: (0, 0)),
        pl.BlockSpec((128, 128), lambda t, i: (0, 0)),
        pl.BlockSpec((1, 128), lambda t, i: (0, 0)),
    ],
    out_specs=[
        pl.BlockSpec((BLK, 128), lambda t, i: (t * NBLK + i, 0)),
        pl.BlockSpec((BLK, ADW), lambda t, i: (t * NBLK + i, 0)),
        pl.BlockSpec((2, 128), lambda t, i: (0, 0)),
    ],
    out_shape=[
        jax.ShapeDtypeStruct((2 * N, 128), jnp.float32),
        jax.ShapeDtypeStruct((2 * N, ADW), jnp.float32),
        jax.ShapeDtypeStruct((2, 128), jnp.float32),
    ],
    scratch_shapes=[pltpu.VMEM((2, 128), jnp.float32)],
)


_sc_mesh = plsc.VectorSubcoreMesh(core_axis_name="c", subcore_axis_name="s")

NBUF = 3
_BUF_SCRATCH = []
for _ in range(NBUF):
    _BUF_SCRATCH += [
        pltpu.VMEM((C,), jnp.int32),
        pltpu.VMEM((C,), jnp.int32),
        pltpu.VMEM((C,), jnp.int32),
        pltpu.VMEM((C,), jnp.int32),
        pltpu.VMEM((C,), jnp.int32),
        pltpu.VMEM((C, 128), jnp.float32),
        pltpu.VMEM((C, ADW), jnp.float32),
        pltpu.VMEM((C, ADW), jnp.float32),
        pltpu.SemaphoreType.DMA,
        pltpu.SemaphoreType.DMA,
        pltpu.SemaphoreType.DMA,
        pltpu.SemaphoreType.DMA,
        pltpu.SemaphoreType.DMA,
        pltpu.SemaphoreType.DMA,
    ]
_NB = 14


@functools.partial(
    pl.kernel,
    out_type=[
        jax.ShapeDtypeStruct((2 * N, 128), jnp.float32),
        jax.ShapeDtypeStruct((2 * N, ADW), jnp.float32),
    ],
    mesh=_sc_mesh,
    scratch_types=[
        pltpu.VMEM_SHARED((N, 128), jnp.float32),
        pltpu.VMEM_SHARED((N, ADW), jnp.float32),
    ] + _BUF_SCRATCH,
    compiler_params=pltpu.CompilerParams(use_tc_tiling_on_sc=False,
                                         needs_layout_passes=False),
)
def _sc_edges(ei_hbm, h_hbm, asad_hbm, msg_hbm, den_hbm,
              accm_sh, accd_sh, *bufs):
    c = lax.axis_index("c")
    s = lax.axis_index("s")
    cn = c * N
    row0 = s * RPS
    B = [bufs[i * _NB:(i + 1) * _NB] for i in range(NBUF)]

    def issue_idx(t, b):
        base = ebase + t * C
        pltpu.async_copy(ei_hbm.at[0, pl.ds(base, C)], B[b][0], B[b][8])
        pltpu.async_copy(ei_hbm.at[1, pl.ds(base, C)], B[b][1], B[b][8])

    def wait_idx(b):
        pltpu.make_async_copy(ei_hbm.at[0, pl.ds(0, C)], B[b][0],
                              B[b][8]).wait()
        pltpu.make_async_copy(ei_hbm.at[1, pl.ds(0, C)], B[b][1],
                              B[b][8]).wait()

    def adjust(b):
        src_v, dst_v, soff, doff, draw = B[b][:5]

        def adj(g, cr):
            sl = pl.ds(g * 16, 16)
            sv = src_v[sl]
            dv = dst_v[sl]
            soff[sl] = sv + cn
            doff[sl] = dv + cn
            draw[sl] = dv
            return cr

        lax.fori_loop(0, C // 16, adj, 0)

    def issue_gather(b):
        pltpu.async_copy(h_hbm.at[B[b][2]], B[b][5], B[b][9])
        pltpu.async_copy(asad_hbm.at[B[b][2]], B[b][6], B[b][10])
        pltpu.async_copy(asad_hbm.at[B[b][3]], B[b][7], B[b][11])

    def wait_gather(b):
        pltpu.make_async_copy(h_hbm.at[B[b][2]], B[b][5], B[b][9]).wait()
        pltpu.make_async_copy(asad_hbm.at[B[b][2]], B[b][6],
                              B[b][10]).wait()
        pltpu.make_async_copy(asad_hbm.at[B[b][3]], B[b][7],
                              B[b][11]).wait()

    def issue_scatter(b):
        pltpu.async_copy(B[b][5], accm_sh.at[B[b][4]], B[b][12], add=True)
        pltpu.async_copy(B[b][7], accd_sh.at[B[b][4]], B[b][13], add=True)

    def wait_scatter(b):
        pltpu.make_async_copy(B[b][5], accm_sh.at[B[b][4]], B[b][12]).wait()
        pltpu.make_async_copy(B[b][7], accd_sh.at[B[b][4]], B[b][13]).wait()

    def compute(b):
        rows, asv, adv = B[b][5], B[b][6], B[b][7]

        def exp_poly(z):
            y = z * 1.4426950408889634
            ni = y.astype(jnp.int32)
            nf = ni.astype(jnp.float32)
            n = ni - jnp.where(y < nf, 1, 0)
            fr = y - n.astype(jnp.float32)
            p = jnp.float32(0.0018943794234292928)
            for ck in (0.008940582529284601, 0.05587655686901505,
                       0.24013169187194985, 0.6931567766988557,
                       0.9999997696337073):
                p = p * fr + jnp.float32(ck)
            bits = (n + 127) << 23
            return p * lax.bitcast_convert_type(bits, jnp.float32)

        def wgroup(g, cr):
            eg = g * 16 + lax.iota(jnp.int32, 16)
            zs = []
            for k in range(H):
                as_k = plsc.load_gather(
                    asv, [eg, jnp.full((16,), k, jnp.int32)])
                ad_k = plsc.load_gather(
                    adv, [eg, jnp.full((16,), H + k, jnp.int32)])
                zs.append(as_k + ad_k)
            ws = [exp_poly(jnp.maximum(z, 0.2 * z)) for z in zs]
            for k in range(H):
                plsc.store_scatter(
                    adv, [eg, jnp.full((16,), k, jnp.int32)], ws[k])
            return cr

        lax.fori_loop(0, C // 16, wgroup, 0)

        def edge_body(e, cr):
            wv = adv[e, pl.ds(0, 16)]
            for k in range(H):
                sl = pl.ds(k * HD, HD)
                rows[e, sl] = rows[e, sl] * wv[k]
            return cr

        lax.fori_loop(0, C, edge_body, 0, unroll=2)

    rows0_v = B[0][5]

    def zero_row(e, cr):
        for k in range(128 // 16):
            rows0_v[e, pl.ds(k * 16, 16)] = jnp.zeros((16,), jnp.float32)
        return cr

    lax.fori_loop(0, C, zero_row, 0)

    nfull = RPS // C
    nrem = RPS - nfull * C

    def zcp(j, cr):
        pltpu.sync_copy(rows0_v, accm_sh.at[pl.ds(row0 + j * C, C)])
        return cr

    lax.fori_loop(0, nfull, zcp, 0)
    pltpu.sync_copy(rows0_v.at[pl.ds(0, nrem)],
                    accm_sh.at[pl.ds(row0 + nfull * C, nrem)])

    def zcpd(j, cr):
        pltpu.sync_copy(rows0_v.at[pl.ds(0, C), pl.ds(0, ADW)],
                        accd_sh.at[pl.ds(row0 + j * C, C)])
        return cr

    lax.fori_loop(0, nfull, zcpd, 0)
    pltpu.sync_copy(rows0_v.at[pl.ds(0, nrem), pl.ds(0, ADW)],
                    accd_sh.at[pl.ds(row0 + nfull * C, nrem)])
    plsc.subcore_barrier()

    ebase = s * EPS
    issue_idx(0, 0)
    wait_idx(0)
    adjust(0)
    issue_gather(0)
    issue_idx(1, 1)
    issue_idx(2, 2)

    NJ = NCHUNK // NBUF

    def loop_j(j, carry):
        t0 = j * NBUF
        for u in range(NBUF):
            t = t0 + u
            b = u
            bb = (u + 1) % NBUF
            wait_idx(bb)

            @pl.when(t >= 2)
            def _ws():
                wait_scatter(bb)

            adjust(bb)
            issue_gather(bb)

            @pl.when(t + 2 < NCHUNK)
            def _ii():
                issue_idx(t + 2, (u + 2) % NBUF)

            wait_gather(b)
            compute(b)
            issue_scatter(b)
        return carry

    lax.fori_loop(0, NJ, loop_j, 0)
    wait_gather(0)
    compute(0)
    issue_scatter(0)

    wait_scatter(1)
    wait_scatter(2)
    wait_scatter(0)
    plsc.subcore_barrier()

    pltpu.sync_copy(accm_sh.at[pl.ds(row0, RPS)],
                    msg_hbm.at[pl.ds(cn + row0, RPS)])
    pltpu.sync_copy(accd_sh.at[pl.ds(row0, RPS)],
                    den_hbm.at[pl.ds(cn + row0, RPS)])


def _k2_body(m0_ref, m1_ref, d0_ref, d1_ref, h0_ref, h1_ref, tsc_ref,
             kW_ref, kb_ref, q_ref, lW_ref, lb_ref, fW_ref, fb_ref,
             out_ref, score_ref):
    p = pl.program_id(0)
    i = pl.program_id(1)
    Eb = _head_selector(0)

    def norm(m_ref, d_ref):
        den128 = jnp.dot(d_ref[...], Eb, preferred_element_type=jnp.float32)
        return jnp.maximum(m_ref[...] / (den128 + 1e-16), 0.0)

    f00 = norm(m0_ref, d0_ref)
    f11 = norm(m1_ref, d1_ref)
    kW = kW_ref[...]
    kb = kb_ref[...]

    @pl.when(p == 0)
    def _phase0():
        @pl.when(i == 0)
        def _init():
            score_ref[...] = jnp.zeros_like(score_ref)

        p00 = jnp.sum(
            jnp.tanh(jnp.dot(f00, kW, preferred_element_type=jnp.float32)
                     + kb), axis=0, keepdims=True)
        p11 = jnp.sum(
            jnp.tanh(jnp.dot(f11, kW, preferred_element_type=jnp.float32)
                     + kb), axis=0, keepdims=True)
        score_ref[0:1, :] += p00
        score_ref[1:2, :] += p11

    @pl.when(p == 1)
    def _phase1():
        f01 = jnp.maximum(h0_ref[...], 0.0)
        f10 = jnp.maximum(h1_ref[...], 0.0)
        qv = q_ref[...] * (1.0 / N)
        s00 = jnp.sum(qv * score_ref[0:1, :])
        s11 = jnp.sum(qv * score_ref[1:2, :])
        s10 = jnp.sum(qv * tsc_ref[0:1, :])
        s01 = jnp.sum(qv * tsc_ref[1:2, :])
        e00 = jnp.exp(s00)
        e10 = jnp.exp(s10)
        e11 = jnp.exp(s11)
        e01 = jnp.exp(s01)
        w00 = e00 / (e00 + e10)
        w10 = 1.0 - w00
        w11 = e11 / (e11 + e01)
        w01 = 1.0 - w11
        lW = lW_ref[...]
        lb = lb_ref[...]
        agg0 = w00 * f00 + w10 * f10
        agg1 = w11 * f11 + w01 * f01
        o0 = jnp.dot(agg0, lW, preferred_element_type=jnp.float32) + lb
        o1 = jnp.dot(agg1, lW, preferred_element_type=jnp.float32) + lb
        ob = jnp.concatenate([o0, o1], axis=1)
        out_ref[...] = (jnp.dot(ob, fW_ref[...],
                                preferred_element_type=jnp.float32)
                        + fb_ref[...])


_k2_call = pl.pallas_call(
    _k2_body,
    grid=(2, NBLK),
    in_specs=[
        pl.BlockSpec((BLK, 128), lambda p, i: (i, 0)),
        pl.BlockSpec((BLK, 128), lambda p, i: (NBLK + i, 0)),
        pl.BlockSpec((BLK, ADW), lambda p, i: (i, 0)),
        pl.BlockSpec((BLK, ADW), lambda p, i: (NBLK + i, 0)),
        pl.BlockSpec((BLK, 128), lambda p, i: (i, 0)),
        pl.BlockSpec((BLK, 128), lambda p, i: (NBLK + i, 0)),
        pl.BlockSpec((2, 128), lambda p, i: (0, 0)),
        pl.BlockSpec((128, 128), lambda p, i: (0, 0)),
        pl.BlockSpec((1, 128), lambda p, i: (0, 0)),
        pl.BlockSpec((1, 128), lambda p, i: (0, 0)),
        pl.BlockSpec((128, 128), lambda p, i: (0, 0)),
        pl.BlockSpec((1, 128), lambda p, i: (0, 0)),
        pl.BlockSpec((256, 64), lambda p, i: (0, 0)),
        pl.BlockSpec((1, 64), lambda p, i: (0, 0)),
    ],
    out_specs=pl.BlockSpec((BLK, 64), lambda p, i: (i, 0)),
    out_shape=jax.ShapeDtypeStruct((N, 64), jnp.float32),
    scratch_shapes=[pltpu.VMEM((2, 128), jnp.float32)],
)


def kernel(x, edge_index, proj0_W, proj0_b, proj1_W, proj1_b,
           att_src_00, att_dst_00, att_src_11, att_dst_11,
           att_src_01, att_dst_01, att_src_10, att_dst_10,
           k_lin_W, k_lin_b, q, lin_W, lin_b, fc_W, fc_b):
    p1w = jnp.concatenate([proj1_W, jnp.zeros((1, 128), jnp.float32)], 0)
    pb = jnp.stack([proj0_b, proj1_b])
    h, asad, tsc = _k1_call(
        x, proj0_W, p1w, pb,
        att_src_00.reshape(1, 128), att_dst_00.reshape(1, 128),
        att_src_11.reshape(1, 128), att_dst_11.reshape(1, 128),
        k_lin_W, k_lin_b.reshape(1, 128))
    msg, den = _sc_edges(edge_index, h, asad)
    out = _k2_call(msg, msg, den, den, h, h, tsc,
                   k_lin_W, k_lin_b.reshape(1, 128), q.reshape(1, 128),
                   lin_W, lin_b.reshape(1, 128), fc_W, fc_b.reshape(1, 64))
    return out
